# baseline (speedup 1.0000x reference)
import functools

import jax
import jax.numpy as jnp
from jax import lax
from jax.experimental import pallas as pl
from jax.experimental.pallas import tpu as pltpu
from jax.experimental.pallas import tpu_sc as plsc

NC = 2
NS = 16
NW = NC * NS
L = 16

BATCH = 16384
HIST = 20
DIM = 64
B = BATCH * HIST
VPAIR = 500000
B_PER_W = B // NW
OUT_PER_W = B_PER_W // 2
CHUNK = 256
OCHUNK = CHUNK // 2
NBUF = 2
NSTEPS = B_PER_W // CHUNK
NROUNDS = NSTEPS // NBUF


@functools.partial(
    pl.kernel,
    out_type=jax.ShapeDtypeStruct((B // 2, 2 * DIM), jnp.float32),
    mesh=plsc.VectorSubcoreMesh(core_axis_name="c", subcore_axis_name="s"),
    scratch_types=[
        pltpu.VMEM((B_PER_W,), jnp.int32),
        [pltpu.VMEM((CHUNK,), jnp.int32) for _ in range(NBUF)],
        [pltpu.VMEM((CHUNK,), jnp.int32) for _ in range(NBUF)],
        pltpu.VMEM((NBUF, CHUNK, 2 * DIM), jnp.float32),
        pltpu.VMEM((NBUF, OCHUNK, 2 * DIM), jnp.float32),
        pltpu.SemaphoreType.DMA((NBUF,)),
        pltpu.SemaphoreType.DMA((NBUF,)),
    ],
    compiler_params=pltpu.CompilerParams(use_tc_tiling_on_sc=True),
)
def _embed_kernel(wpair_hbm, idx_hbm, out_hbm, idx_v, gidx_v, off_v, pairs_v,
                  outs_v, gsem, ssem):
    wid = lax.axis_index("s") * NC + lax.axis_index("c")
    base = wid * B_PER_W
    obase = wid * OUT_PER_W

    pltpu.sync_copy(idx_hbm.at[pl.ds(base, B_PER_W)], idx_v)

    def build_and_gather(j, b):
        @pl.loop(0, CHUNK // L)
        def _vec(k):
            v = idx_v[pl.ds(j * CHUNK + k * L, L)]
            gidx_v[b][pl.ds(k * L, L)] = v >> 1
            off_v[b][pl.ds(k * L, L)] = (v & 1) << 6
        pltpu.async_copy(wpair_hbm.at[gidx_v[b]], pairs_v.at[b], gsem.at[b])

    def extract(b):
        @pl.loop(0, OCHUNK // 8)
        def _grp(t):
            offv = off_v[b][pl.ds(t * L, L)]
            for j in range(8):
                m = t * 8 + j
                off_l = offv[2 * j]
                off_r = offv[2 * j + 1]
                for c in range(4):
                    outs_v[b, m, pl.ds(c * L, L)] = (
                        pairs_v[b, 2 * m, pl.ds(off_l + c * L, L)])
                    outs_v[b, m, pl.ds(DIM + c * L, L)] = (
                        pairs_v[b, 2 * m + 1, pl.ds(off_r + c * L, L)])

    for b in range(NBUF):
        build_and_gather(b, b)

    @pl.loop(0, NROUNDS)
    def _round(g):
        j0 = g * NBUF
        for b in range(NBUF):
            @pl.when(g > 0)
            def _():
                pltpu.make_async_copy(
                    outs_v.at[b], out_hbm.at[pl.ds(0, OCHUNK)], ssem.at[b]
                ).wait()
            pltpu.make_async_copy(
                wpair_hbm.at[gidx_v[b]], pairs_v.at[b], gsem.at[b]
            ).wait()
            extract(b)
            pltpu.async_copy(
                outs_v.at[b],
                out_hbm.at[pl.ds(obase + (j0 + b) * OCHUNK, OCHUNK)],
                ssem.at[b],
            )
            @pl.when(j0 + b + NBUF < NSTEPS)
            def _():
                build_and_gather(j0 + b + NBUF, b)

    for b in range(NBUF):
        pltpu.make_async_copy(
            outs_v.at[b], out_hbm.at[pl.ds(0, OCHUNK)], ssem.at[b]
        ).wait()


def kernel(input_, weight):
    idx = input_.reshape(-1).astype(jnp.int32)
    wpair = weight.reshape(VPAIR, 2 * DIM)
    out2 = _embed_kernel(wpair, idx)
    return out2.reshape(BATCH, HIST, DIM)
